# Initial kernel scaffold; baseline (speedup 1.0000x reference)
import jax
import jax.numpy as jnp
from jax.experimental import pallas as pl


def kernel(initial_node_representation, edge_index_0, edge_index_1, W_msg_0_0, b_msg_0_0, W_msg_0_1, b_msg_0_1, W_ih_0, W_hh_0, b_ih_0, b_hh_0, W_msg_1_0, b_msg_1_0, W_msg_1_1, b_msg_1_1, W_ih_1, W_hh_1, b_ih_1, b_hh_1):
    raise NotImplementedError("write your pallas kernel here")


# speedup vs baseline: 2.7943x; 2.7943x over previous
import jax
import jax.numpy as jnp
from jax import lax
from jax.experimental import pallas as pl
from jax.experimental.pallas import tpu as pltpu
from jax.experimental.pallas import tpu_sc as plsc

N = 10000
E = 80000
H = 128
NC = 2
NT = 16
CHUNK = 128
NCHUNK = 40
EPT = NCHUNK * CHUNK
EPAD = NT * EPT
NP = 10240
RPT = NP // NT
ZR = 128
PAD_TGT = 10200

_F32 = jnp.float32


def _segsum_body(h_hbm, src_hbm, tgt_hbm, out_hbm,
                 src_v, tgt_v, rows_v, zrow_v, acc, sem):
    c = lax.axis_index("c")
    s = lax.axis_index("s")

    def _zrow(i, carry):
        for j in range(H // 16):
            zrow_v[i, pl.ds(j * 16, 16)] = jnp.zeros((16,), _F32)
        return carry
    lax.fori_loop(0, ZR, _zrow, 0)

    def _zacc(k, carry):
        pltpu.sync_copy(zrow_v, acc.at[pl.ds(s * RPT + k * ZR, ZR)])
        return carry
    lax.fori_loop(0, RPT // ZR, _zacc, 0)
    plsc.subcore_barrier()

    pltpu.sync_copy(src_hbm.at[c, s], src_v)
    pltpu.sync_copy(tgt_hbm.at[c, s], tgt_v)

    def _edge(j, carry):
        pltpu.async_copy(h_hbm.at[src_v.at[j]], rows_v, sem).wait()
        pltpu.sync_copy(rows_v, acc.at[tgt_v.at[j]], add=True)
        return carry
    lax.fori_loop(0, NCHUNK, _edge, 0)
    plsc.subcore_barrier()

    pltpu.sync_copy(acc.at[pl.ds(s * RPT, RPT)],
                    out_hbm.at[c, pl.ds(s * RPT, RPT)])


def _build_segsum():
    return pl.kernel(
        _segsum_body,
        out_type=jax.ShapeDtypeStruct((NC, NP, H), _F32),
        mesh=plsc.VectorSubcoreMesh(core_axis_name="c", subcore_axis_name="s",
                                    num_cores=NC, num_subcores=NT),
        scratch_types=[
            pltpu.VMEM((NCHUNK, CHUNK), jnp.int32),
            pltpu.VMEM((NCHUNK, CHUNK), jnp.int32),
            pltpu.VMEM((CHUNK, H), _F32),
            pltpu.VMEM((ZR, H), _F32),
            pltpu.VMEM_SHARED((NP, H), _F32),
            pltpu.SemaphoreType.DMA,
        ],
    )


def _deg_body(tgt_hbm, out_hbm, tgt_v, val_v, dacc):
    c = lax.axis_index("c")
    s = lax.axis_index("s")

    def _fill(val):
        def _f(i, carry):
            for j in range(H // 16):
                val_v[i, pl.ds(j * 16, 16)] = jnp.full((16,), val, _F32)
            return carry
        lax.fori_loop(0, ZR, _f, 0)

    _fill(0.0)

    def _zacc(k, carry):
        pltpu.sync_copy(val_v, dacc.at[pl.ds(s * RPT + k * ZR, ZR)])
        return carry
    lax.fori_loop(0, RPT // ZR, _zacc, 0)

    _fill(1.0)
    plsc.subcore_barrier()

    pltpu.sync_copy(tgt_hbm.at[c, s], tgt_v)

    def _edge(j, carry):
        pltpu.sync_copy(val_v, dacc.at[tgt_v.at[j]], add=True)
        return carry
    lax.fori_loop(0, NCHUNK, _edge, 0)
    plsc.subcore_barrier()

    pltpu.sync_copy(dacc.at[pl.ds(s * RPT, RPT)],
                    out_hbm.at[c, pl.ds(s * RPT, RPT)])


def _build_deg():
    return pl.kernel(
        _deg_body,
        out_type=jax.ShapeDtypeStruct((NC, NP, H), _F32),
        mesh=plsc.VectorSubcoreMesh(core_axis_name="c", subcore_axis_name="s",
                                    num_cores=NC, num_subcores=NT),
        scratch_types=[
            pltpu.VMEM((NCHUNK, CHUNK), jnp.int32),
            pltpu.VMEM((ZR, H), _F32),
            pltpu.VMEM_SHARED((NP, H), _F32),
        ],
    )


BN = 1000


def _gru_body(g_ref, h_ref, d_ref,
              w0t_ref, w1t_ref, b0_ref, b1_ref,
              wiht_ref, whht_ref, bih_ref, bhh_ref, out_ref, outr_ref):
    h = h_ref[...]
    hi = lax.Precision.HIGHEST
    lo = lax.Precision.DEFAULT
    inc = jnp.dot(g_ref[0], w0t_ref[...], precision=hi,
                  preferred_element_type=_F32)
    inc += jnp.dot(g_ref[1], w1t_ref[...], precision=hi,
                   preferred_element_type=_F32)
    inc += d_ref[0] * b0_ref[...]
    inc += d_ref[1] * b1_ref[...]
    gi = jnp.dot(inc, wiht_ref[...], precision=lo,
                 preferred_element_type=_F32) + bih_ref[...]
    gh = jnp.dot(h, whht_ref[...], precision=lo,
                 preferred_element_type=_F32) + bhh_ref[...]
    r = jax.nn.sigmoid(gi[:, 0:H] + gh[:, 0:H])
    z = jax.nn.sigmoid(gi[:, H:2 * H] + gh[:, H:2 * H])
    n = jnp.tanh(gi[:, 2 * H:] + r * gh[:, 2 * H:])
    out = (1.0 - z) * n + z * h
    out_ref[...] = out
    outr_ref[...] = out.astype(jnp.bfloat16).astype(_F32)


def _row_spec(w):
    return pl.BlockSpec((BN, w), lambda i: (i, 0))


def _full_spec(r, w):
    return pl.BlockSpec((r, w), lambda i: (0, 0))


_gru = pl.pallas_call(
    _gru_body,
    grid=(N // BN,),
    in_specs=[
        pl.BlockSpec((NC, BN, H), lambda i: (0, i, 0)),
        _row_spec(H),
        pl.BlockSpec((NC, BN, H), lambda i: (0, i, 0)),
        _full_spec(H, H), _full_spec(H, H),
        _full_spec(1, H), _full_spec(1, H),
        _full_spec(H, 3 * H), _full_spec(H, 3 * H),
        _full_spec(1, 3 * H), _full_spec(1, 3 * H),
    ],
    out_specs=(_row_spec(H), _row_spec(H)),
    out_shape=(jax.ShapeDtypeStruct((N, H), _F32),
               jax.ShapeDtypeStruct((N, H), _F32)),
)


def kernel(initial_node_representation, edge_index_0, edge_index_1,
           W_msg_0_0, b_msg_0_0, W_msg_0_1, b_msg_0_1,
           W_ih_0, W_hh_0, b_ih_0, b_hh_0,
           W_msg_1_0, b_msg_1_0, W_msg_1_1, b_msg_1_1,
           W_ih_1, W_hh_1, b_ih_1, b_hh_1):
    pad_s = jnp.zeros((EPAD - E,), jnp.int32)
    pad_t = jnp.full((EPAD - E,), PAD_TGT, jnp.int32)
    src = jnp.stack([jnp.concatenate([edge_index_0[:, 0], pad_s]),
                     jnp.concatenate([edge_index_1[:, 0], pad_s])])
    src = src.reshape(NC, NT, NCHUNK, CHUNK)
    tgt = jnp.stack([jnp.concatenate([edge_index_0[:, 1], pad_t]),
                     jnp.concatenate([edge_index_1[:, 1], pad_t])])
    tgt = tgt.reshape(NC, NT, NCHUNK, CHUNK)

    segsum = _build_segsum()
    d = _build_deg()(tgt)

    def _rnd(w):
        return lax.optimization_barrier(w.astype(jnp.bfloat16)).astype(_F32)

    layers = [
        (_rnd(W_msg_0_0.T), _rnd(W_msg_0_1.T), b_msg_0_0.reshape(1, H),
         b_msg_0_1.reshape(1, H), W_ih_0.T, W_hh_0.T,
         b_ih_0.reshape(1, 3 * H), b_hh_0.reshape(1, 3 * H)),
        (_rnd(W_msg_1_0.T), _rnd(W_msg_1_1.T), b_msg_1_0.reshape(1, H),
         b_msg_1_1.reshape(1, H), W_ih_1.T, W_hh_1.T,
         b_ih_1.reshape(1, 3 * H), b_hh_1.reshape(1, 3 * H)),
    ]

    h = initial_node_representation
    hr = _rnd(h)
    for (w0t, w1t, b0, b1, wiht, whht, bih, bhh) in layers:
        for _ in range(2):
            g = segsum(hr, src, tgt)
            h, hr = _gru(g, h, d, w0t, w1t, b0, b1,
                         wiht, whht, bih, bhh)
    return h


# speedup vs baseline: 3.0655x; 1.0971x over previous
import jax
import jax.numpy as jnp
from jax import lax
from jax.experimental import pallas as pl
from jax.experimental.pallas import tpu as pltpu
from jax.experimental.pallas import tpu_sc as plsc

N = 10000
E = 80000
H = 128
NC = 2
NT = 16
CHUNK = 128
NCHUNK = 40
EPT = NCHUNK * CHUNK
EPAD = NT * EPT
NP = 10240
RPT = NP // NT
ZR = 128
PAD_TGT = 10200

_F32 = jnp.float32


def _segsum_body(h_hbm, src_hbm, tgt_hbm, out_hbm,
                 src_v, tgt_v, rows0_v, rows1_v, acc, sem0, sem1):
    c = lax.axis_index("c")
    s = lax.axis_index("s")

    def _zrow(i, carry):
        for j in range(H // 16):
            rows0_v[i, pl.ds(j * 16, 16)] = jnp.zeros((16,), _F32)
        return carry
    lax.fori_loop(0, ZR, _zrow, 0)

    def _zacc(k, carry):
        pltpu.sync_copy(rows0_v, acc.at[pl.ds(s * RPT + k * ZR, ZR)])
        return carry
    lax.fori_loop(0, RPT // ZR, _zacc, 0)
    plsc.subcore_barrier()

    pltpu.sync_copy(src_hbm.at[c, s], src_v)
    pltpu.sync_copy(tgt_hbm.at[c, s], tgt_v)

    pltpu.async_copy(h_hbm.at[src_v.at[0]], rows0_v, sem0)

    def _edge2(i, carry):
        j0 = 2 * i
        d1 = pltpu.async_copy(h_hbm.at[src_v.at[j0 + 1]], rows1_v, sem1)
        pltpu.make_async_copy(h_hbm.at[src_v.at[0]], rows0_v, sem0).wait()
        pltpu.sync_copy(rows0_v, acc.at[tgt_v.at[j0]], add=True)

        @pl.when(i < NCHUNK // 2 - 1)
        def _():
            pltpu.async_copy(h_hbm.at[src_v.at[j0 + 2]], rows0_v, sem0)

        d1.wait()
        pltpu.sync_copy(rows1_v, acc.at[tgt_v.at[j0 + 1]], add=True)
        return carry
    lax.fori_loop(0, NCHUNK // 2, _edge2, 0)
    plsc.subcore_barrier()

    pltpu.sync_copy(acc.at[pl.ds(s * RPT, RPT)],
                    out_hbm.at[c, pl.ds(s * RPT, RPT)])


def _build_segsum():
    return pl.kernel(
        _segsum_body,
        out_type=jax.ShapeDtypeStruct((NC, NP, H), _F32),
        mesh=plsc.VectorSubcoreMesh(core_axis_name="c", subcore_axis_name="s",
                                    num_cores=NC, num_subcores=NT),
        scratch_types=[
            pltpu.VMEM((NCHUNK, CHUNK), jnp.int32),
            pltpu.VMEM((NCHUNK, CHUNK), jnp.int32),
            pltpu.VMEM((CHUNK, H), _F32),
            pltpu.VMEM((CHUNK, H), _F32),
            pltpu.VMEM_SHARED((NP, H), _F32),
            pltpu.SemaphoreType.DMA,
            pltpu.SemaphoreType.DMA,
        ],
    )


def _deg_body(tgt_hbm, out_hbm, tgt_v, val_v, dacc):
    c = lax.axis_index("c")
    s = lax.axis_index("s")

    def _fill(val):
        def _f(i, carry):
            for j in range(H // 16):
                val_v[i, pl.ds(j * 16, 16)] = jnp.full((16,), val, _F32)
            return carry
        lax.fori_loop(0, ZR, _f, 0)

    _fill(0.0)

    def _zacc(k, carry):
        pltpu.sync_copy(val_v, dacc.at[pl.ds(s * RPT + k * ZR, ZR)])
        return carry
    lax.fori_loop(0, RPT // ZR, _zacc, 0)

    _fill(1.0)
    plsc.subcore_barrier()

    pltpu.sync_copy(tgt_hbm.at[c, s], tgt_v)

    def _edge(j, carry):
        pltpu.sync_copy(val_v, dacc.at[tgt_v.at[j]], add=True)
        return carry
    lax.fori_loop(0, NCHUNK, _edge, 0)
    plsc.subcore_barrier()

    pltpu.sync_copy(dacc.at[pl.ds(s * RPT, RPT)],
                    out_hbm.at[c, pl.ds(s * RPT, RPT)])


def _build_deg():
    return pl.kernel(
        _deg_body,
        out_type=jax.ShapeDtypeStruct((NC, NP, H), _F32),
        mesh=plsc.VectorSubcoreMesh(core_axis_name="c", subcore_axis_name="s",
                                    num_cores=NC, num_subcores=NT),
        scratch_types=[
            pltpu.VMEM((NCHUNK, CHUNK), jnp.int32),
            pltpu.VMEM((ZR, H), _F32),
            pltpu.VMEM_SHARED((NP, H), _F32),
        ],
    )


BN = 1000


def _gru_body(g_ref, h_ref, d_ref,
              w0t_ref, w1t_ref, b0_ref, b1_ref,
              wiht_ref, whht_ref, bih_ref, bhh_ref, out_ref, outr_ref):
    h = h_ref[...]
    hi = lax.Precision.HIGHEST
    lo = lax.Precision.DEFAULT
    inc = jnp.dot(g_ref[0], w0t_ref[...], precision=hi,
                  preferred_element_type=_F32)
    inc += jnp.dot(g_ref[1], w1t_ref[...], precision=hi,
                   preferred_element_type=_F32)
    inc += d_ref[0] * b0_ref[...]
    inc += d_ref[1] * b1_ref[...]
    gi = jnp.dot(inc, wiht_ref[...], precision=lo,
                 preferred_element_type=_F32) + bih_ref[...]
    gh = jnp.dot(h, whht_ref[...], precision=lo,
                 preferred_element_type=_F32) + bhh_ref[...]
    r = jax.nn.sigmoid(gi[:, 0:H] + gh[:, 0:H])
    z = jax.nn.sigmoid(gi[:, H:2 * H] + gh[:, H:2 * H])
    n = jnp.tanh(gi[:, 2 * H:] + r * gh[:, 2 * H:])
    out = (1.0 - z) * n + z * h
    out_ref[...] = out
    outr_ref[...] = out.astype(jnp.bfloat16).astype(_F32)


def _row_spec(w):
    return pl.BlockSpec((BN, w), lambda i: (i, 0))


def _full_spec(r, w):
    return pl.BlockSpec((r, w), lambda i: (0, 0))


_gru = pl.pallas_call(
    _gru_body,
    grid=(N // BN,),
    in_specs=[
        pl.BlockSpec((NC, BN, H), lambda i: (0, i, 0)),
        _row_spec(H),
        pl.BlockSpec((NC, BN, H), lambda i: (0, i, 0)),
        _full_spec(H, H), _full_spec(H, H),
        _full_spec(1, H), _full_spec(1, H),
        _full_spec(H, 3 * H), _full_spec(H, 3 * H),
        _full_spec(1, 3 * H), _full_spec(1, 3 * H),
    ],
    out_specs=(_row_spec(H), _row_spec(H)),
    out_shape=(jax.ShapeDtypeStruct((N, H), _F32),
               jax.ShapeDtypeStruct((N, H), _F32)),
)


def kernel(initial_node_representation, edge_index_0, edge_index_1,
           W_msg_0_0, b_msg_0_0, W_msg_0_1, b_msg_0_1,
           W_ih_0, W_hh_0, b_ih_0, b_hh_0,
           W_msg_1_0, b_msg_1_0, W_msg_1_1, b_msg_1_1,
           W_ih_1, W_hh_1, b_ih_1, b_hh_1):
    pad_s = jnp.zeros((EPAD - E,), jnp.int32)
    pad_t = jnp.full((EPAD - E,), PAD_TGT, jnp.int32)
    src = jnp.stack([jnp.concatenate([edge_index_0[:, 0], pad_s]),
                     jnp.concatenate([edge_index_1[:, 0], pad_s])])
    src = src.reshape(NC, NT, NCHUNK, CHUNK)
    tgt = jnp.stack([jnp.concatenate([edge_index_0[:, 1], pad_t]),
                     jnp.concatenate([edge_index_1[:, 1], pad_t])])
    tgt = tgt.reshape(NC, NT, NCHUNK, CHUNK)

    segsum = _build_segsum()
    d = _build_deg()(tgt)

    def _rnd(w):
        return lax.optimization_barrier(w.astype(jnp.bfloat16)).astype(_F32)

    layers = [
        (_rnd(W_msg_0_0.T), _rnd(W_msg_0_1.T), b_msg_0_0.reshape(1, H),
         b_msg_0_1.reshape(1, H), W_ih_0.T, W_hh_0.T,
         b_ih_0.reshape(1, 3 * H), b_hh_0.reshape(1, 3 * H)),
        (_rnd(W_msg_1_0.T), _rnd(W_msg_1_1.T), b_msg_1_0.reshape(1, H),
         b_msg_1_1.reshape(1, H), W_ih_1.T, W_hh_1.T,
         b_ih_1.reshape(1, 3 * H), b_hh_1.reshape(1, 3 * H)),
    ]

    h = initial_node_representation
    hr = _rnd(h)
    for (w0t, w1t, b0, b1, wiht, whht, bih, bhh) in layers:
        for _ in range(2):
            g = segsum(hr, src, tgt)
            h, hr = _gru(g, h, d, w0t, w1t, b0, b1,
                         wiht, whht, bih, bhh)
    return h
